# Initial kernel scaffold; baseline (speedup 1.0000x reference)
import jax
import jax.numpy as jnp
from jax.experimental import pallas as pl


def kernel(box, cls, center, points, revise):
    raise NotImplementedError("write your pallas kernel here")


# speedup vs baseline: 1.0095x; 1.0095x over previous
import jax
import jax.numpy as jnp
from jax.experimental import pallas as pl

N, C, H, W = 8, 80, 128, 128
HW = H * W
K = 1000
THRESH = 0.05


def _scores_body(cls_ref, center_ref, out_ref):
    s = 1.0 / (1.0 + jnp.exp(-cls_ref[...]))
    ctr = 1.0 / (1.0 + jnp.exp(-center_ref[...]))
    out_ref[...] = jnp.where(s > THRESH, s * ctr, 0.0)


def _scores(cls_hwc, center_hw1):
    BLK = 2048
    return pl.pallas_call(
        _scores_body,
        grid=(N, HW // BLK),
        in_specs=[
            pl.BlockSpec((1, BLK, C), lambda n, i: (n, i, 0)),
            pl.BlockSpec((1, BLK, 1), lambda n, i: (n, i, 0)),
        ],
        out_specs=pl.BlockSpec((1, BLK, C), lambda n, i: (n, i, 0)),
        out_shape=jax.ShapeDtypeStruct((N, HW, C), jnp.float32),
    )(cls_hwc, center_hw1)


def kernel(box, cls, center, points, revise):
    cls_hwc = jnp.transpose(cls, (0, 2, 3, 1)).reshape(N, HW, C)
    center_hw1 = center.reshape(N, HW, 1)
    scores = _scores(cls_hwc, center_hw1)

    flat = scores.reshape(N, HW * C)
    topv, topi = jax.lax.top_k(flat, K)
    loc = topi // C
    clsid = topi % C + 1

    box_r = jnp.transpose(box.reshape(N, 4, HW), (0, 2, 1))
    revise_r = revise.reshape(N, HW, 2)
    per_box = jnp.take_along_axis(box_r, loc[:, :, None], axis=1)
    pts = points[loc] + jnp.take_along_axis(revise_r, loc[:, :, None], axis=1)

    detections = jnp.stack(
        [
            pts[..., 0] - per_box[..., 0],
            pts[..., 1] - per_box[..., 1],
            pts[..., 0] + per_box[..., 2],
            pts[..., 1] + per_box[..., 3],
        ],
        axis=-1,
    )
    return detections, clsid, topv


# speedup vs baseline: 2.6770x; 2.6518x over previous
import functools

import jax
import jax.numpy as jnp
from jax import lax
from jax.experimental import pallas as pl
from jax.experimental.pallas import tpu as pltpu
from jax.experimental.pallas import tpu_sc as plsc

N, C, H, W = 8, 80, 128, 128
HW = H * W
FL = HW * C
K = 1000
THRESH = 0.05
NB = 512
NBF = float(NB)
NCORE = 2
NSUB = 16
BPC = N // NCORE
CH = FL // NSUB
NV = CH // 16
PER_W = 1024
SLOTS = NSUB * PER_W
KP = 1024
PP = KP // 4
OFF_PTS = N * 4 * HW
OFF_REV = OFF_PTS + HW * 2


def _scores_body(cls_ref, center_ref, out_ref):
    s = 1.0 / (1.0 + jnp.exp(-cls_ref[...]))
    ctr = 1.0 / (1.0 + jnp.exp(-center_ref[...]))
    out_ref[...] = jnp.where(s > THRESH, s * ctr, 0.0)


def _scores(cls_hwc, center_hw1):
    BLK = 2048
    return pl.pallas_call(
        _scores_body,
        grid=(N, HW // BLK),
        in_specs=[
            pl.BlockSpec((1, BLK, C), lambda n, i: (n, i, 0)),
            pl.BlockSpec((1, BLK, 1), lambda n, i: (n, i, 0)),
        ],
        out_specs=pl.BlockSpec((1, BLK, C), lambda n, i: (n, i, 0)),
        out_shape=jax.ShapeDtypeStruct((N, HW, C), jnp.float32),
    )(cls_hwc, center_hw1)


_sel_mesh = plsc.VectorSubcoreMesh(core_axis_name="c", subcore_axis_name="s")


@functools.partial(
    pl.kernel,
    out_type=(
        jax.ShapeDtypeStruct((N, SLOTS), jnp.float32),
        jax.ShapeDtypeStruct((N, SLOTS), jnp.int32),
    ),
    mesh=_sel_mesh,
    compiler_params=pltpu.CompilerParams(needs_layout_passes=False),
    scratch_types=[
        pltpu.VMEM((CH,), jnp.float32),
        pltpu.VMEM((NB * 16,), jnp.int32),
        pltpu.VMEM((NB,), jnp.int32),
        pltpu.VMEM((NB,), jnp.int32),
        pltpu.VMEM((PER_W,), jnp.float32),
        pltpu.VMEM((PER_W,), jnp.int32),
        pltpu.VMEM((16,), jnp.int32),
        pltpu.VMEM_SHARED((NSUB, BPC, NB), jnp.int32),
        pltpu.VMEM_SHARED((BPC, 128), jnp.int32),
    ],
)
def _select(scores_ref, slot_v_ref, slot_i_ref,
            win, hist16, histc, histt, cval, cidx, tbuf, sh_hist, sh_thr):
    cid = lax.axis_index("c")
    sid = lax.axis_index("s")
    lane = lax.iota(jnp.int32, 16)
    lane_nb = lane * NB
    ones = jnp.ones((16,), jnp.int32)
    zeros16 = jnp.zeros((16,), jnp.int32)
    nb1 = jnp.full((16,), NB - 1, jnp.int32)
    col0 = sid * CH

    def batch_hist(bl, _):
        b = cid * BPC + bl

        def zbody(i, _):
            hist16[pl.ds(i * 16, 16)] = zeros16
            return 0
        lax.fori_loop(0, NB, zbody, 0)

        pltpu.sync_copy(scores_ref.at[b, pl.ds(col0, CH)], win)

        def hbody(i, _):
            v = win[pl.ds(i * 16, 16)]
            bn = lax.min(lax.convert_element_type(v * NBF, jnp.int32), nb1)
            idx = bn + lane_nb
            cur = plsc.load_gather(hist16, [idx])
            plsc.store_scatter(hist16, [idx], cur + ones)
            return 0
        lax.fori_loop(0, NV, hbody, 0)

        def cbody(p, _):
            acc = hist16[pl.ds(p * 16, 16)]
            for l in range(1, 16):
                acc = acc + hist16[pl.ds(l * NB + p * 16, 16)]
            histc[pl.ds(p * 16, 16)] = acc
            return 0
        lax.fori_loop(0, NB // 16, cbody, 0)

        pltpu.sync_copy(histc, sh_hist.at[sid, bl])
        return 0
    lax.fori_loop(0, BPC, batch_hist, 0)
    plsc.subcore_barrier()

    @pl.when(sid < BPC)
    def _():
        bl = sid

        def zbody(i, _):
            histc[pl.ds(i * 16, 16)] = zeros16
            return 0
        lax.fori_loop(0, NB // 16, zbody, 0)

        def sum_w(wkr, _):
            pltpu.sync_copy(sh_hist.at[wkr, bl], histt)

            def abody(p, _):
                histc[pl.ds(p * 16, 16)] = (
                    histc[pl.ds(p * 16, 16)] + histt[pl.ds(p * 16, 16)])
                return 0
            lax.fori_loop(0, NB // 16, abody, 0)
            return 0
        lax.fori_loop(0, NSUB, sum_w, 0)

        def scan_bin(t, carry):
            s_after, jstar = carry
            p = NB // 16 - 1 - t
            chunk = histc[pl.ds(p * 16, 16)]
            ps = plsc.cumsum(chunk)
            total = jnp.sum(chunk)
            suffix = s_after + total - ps + chunk
            cond = suffix >= K
            pc = plsc.all_reduce_population_count(cond)[0]
            jstar = lax.select((pc > 0) & (jstar < 0),
                               p * 16 + pc - 1, jstar)
            return (s_after + total, jstar)
        _, jstar = lax.fori_loop(0, NB // 16, scan_bin,
                                 (jnp.int32(0), jnp.int32(-1)))
        jstar = lax.max(jstar, jnp.int32(0))
        tbuf[pl.ds(0, 16)] = zeros16 + jstar
        pltpu.sync_copy(tbuf, sh_thr.at[bl, pl.ds(0, 16)])
    plsc.subcore_barrier()

    neg1 = jnp.full((16,), -1.0, jnp.float32)
    hw0 = sid * (CH // 80)

    def batch_comp(bl, _):
        b = cid * BPC + bl
        pltpu.sync_copy(sh_thr.at[bl, pl.ds(0, 16)], tbuf)
        thr = tbuf[pl.ds(0, 16)][0]

        def pbody(i, _):
            cval[pl.ds(i * 16, 16)] = neg1
            return 0
        lax.fori_loop(0, PER_W // 16, pbody, 0)

        pltpu.sync_copy(scores_ref.at[b, pl.ds(col0, CH)], win)

        def cbody(i, carry):
            pos, pcb, cs = carry
            v = win[pl.ds(i * 16, 16)]
            bn = lax.min(lax.convert_element_type(v * NBF, jnp.int32), nb1)
            m = bn >= thr
            cnt_v = plsc.all_reduce_population_count(m)
            cnt = cnt_v[0]
            ok = pos <= PER_W - 16

            @pl.when(ok)
            def _():
                plsc.store_compressed(cval.at[pl.ds(pos, 16)], v, mask=m)
                plsc.store_compressed(cidx.at[pl.ds(pos, 16)], pcb + lane,
                                      mask=m)

            pos = pos + lax.select(ok, cnt, jnp.int32(0))
            wrap = cs >= 80 - 16
            pcb = pcb + lax.select(wrap, jnp.int32(64), jnp.int32(16))
            cs = lax.select(wrap, jnp.int32(0), cs + 16)
            return (pos, pcb, cs)
        lax.fori_loop(0, NV, cbody,
                      (jnp.int32(0), (hw0 * 128).astype(jnp.int32),
                       jnp.int32(0)))

        pltpu.sync_copy(cval, slot_v_ref.at[b, pl.ds(sid * PER_W, PER_W)])
        pltpu.sync_copy(cidx, slot_i_ref.at[b, pl.ds(sid * PER_W, PER_W)])
        return 0
    lax.fori_loop(0, BPC, batch_comp, 0)


@functools.partial(
    pl.kernel,
    out_type=(
        jax.ShapeDtypeStruct((N, KP * 4), jnp.float32),
        jax.ShapeDtypeStruct((N, KP), jnp.int32),
    ),
    mesh=_sel_mesh,
    compiler_params=pltpu.CompilerParams(needs_layout_passes=False),
    scratch_types=[
        pltpu.VMEM((PP,), jnp.int32),
        pltpu.VMEM((2, 128), jnp.int32),
        pltpu.VMEM((2, 128), jnp.int32),
        pltpu.VMEM((PP,), jnp.int32),
        pltpu.VMEM((PP,), jnp.int32),
        pltpu.VMEM((16, 128), jnp.int32),
        pltpu.VMEM((16, 128), jnp.float32),
        pltpu.VMEM((KP,), jnp.float32),
        pltpu.SemaphoreType.DMA,
    ],
)
def _gather(slot_i_ref, tslot_ref, src_ref, det_ref, cls_ref,
            tsl, asl, fid, locb, clsb, gi, gv, detv, sem):
    cid = lax.axis_index("c")
    sid = lax.axis_index("s")
    wid = sid * NCORE + cid
    b = wid // 4
    sub = wid - b * 4
    lane = lax.iota(jnp.int32, 16)

    pltpu.sync_copy(tslot_ref.at[b, pl.ds(sub * PP, PP)], tsl)

    def abody(k, _):
        h = k // 8
        off = (k - h * 8) * 16
        asl[h, pl.ds(off, 16)] = tsl[pl.ds(k * 16, 16)] + b * SLOTS
        return 0
    lax.fori_loop(0, 16, abody, 0)

    d0 = pltpu.async_copy(slot_i_ref.at[asl.at[0]], fid.at[0], sem)
    d1 = pltpu.async_copy(slot_i_ref.at[asl.at[1]], fid.at[1], sem)
    d0.wait()
    d1.wait()

    def lbody(k, _):
        h = k // 8
        off = (k - h * 8) * 16
        pc = fid[h, pl.ds(off, 16)]
        locb[pl.ds(k * 16, 16)] = lax.shift_right_logical(pc, 7)
        clsb[pl.ds(k * 16, 16)] = (pc & 127) + 1
        return 0
    lax.fori_loop(0, 16, lbody, 0)

    def make_rows(g, base_fn):
        def rbody(k, _):
            h = k // 8
            off = (k - h * 8) * 16
            loc = locb[pl.ds(k * 16, 16)]
            gi[2 * g + h, pl.ds(off, 16)] = base_fn(loc)
            return 0
        lax.fori_loop(0, 16, rbody, 0)

    box_base = b * 4 * HW
    make_rows(0, lambda loc: box_base + loc)
    make_rows(1, lambda loc: box_base + HW + loc)
    make_rows(2, lambda loc: box_base + 2 * HW + loc)
    make_rows(3, lambda loc: box_base + 3 * HW + loc)
    make_rows(4, lambda loc: OFF_PTS + loc * 2)
    make_rows(5, lambda loc: OFF_PTS + loc * 2 + 1)
    make_rows(6, lambda loc: OFF_REV + (b * HW + loc) * 2)
    make_rows(7, lambda loc: OFF_REV + (b * HW + loc) * 2 + 1)

    descs = []
    for r in range(16):
        descs.append(pltpu.async_copy(src_ref.at[gi.at[r]], gv.at[r], sem))
    for d in descs:
        d.wait()

    def dbody(k, _):
        h = k // 8
        off = (k - h * 8) * 16
        bx0 = gv[0 + h, pl.ds(off, 16)]
        bx1 = gv[2 + h, pl.ds(off, 16)]
        bx2 = gv[4 + h, pl.ds(off, 16)]
        bx3 = gv[6 + h, pl.ds(off, 16)]
        px = gv[8 + h, pl.ds(off, 16)] + gv[12 + h, pl.ds(off, 16)]
        py = gv[10 + h, pl.ds(off, 16)] + gv[14 + h, pl.ds(off, 16)]
        pidx = (k * 16 + lane) * 4
        plsc.store_scatter(detv, [pidx], px - bx0)
        plsc.store_scatter(detv, [pidx + 1], py - bx1)
        plsc.store_scatter(detv, [pidx + 2], px + bx2)
        plsc.store_scatter(detv, [pidx + 3], py + bx3)
        return 0
    lax.fori_loop(0, 16, dbody, 0)

    pltpu.sync_copy(detv, det_ref.at[b, pl.ds(sub * PP * 4, PP * 4)])
    pltpu.sync_copy(clsb, cls_ref.at[b, pl.ds(sub * PP, PP)])


def kernel(box, cls, center, points, revise):
    cls_hwc = jnp.transpose(cls, (0, 2, 3, 1)).reshape(N, HW, C)
    center_hw1 = center.reshape(N, HW, 1)
    scores = _scores(cls_hwc, center_hw1).reshape(N, FL)

    slot_v, slot_i = _select(scores)
    topv, tslot = lax.top_k(slot_v, K)
    tslot_pad = jnp.pad(tslot, ((0, 0), (0, KP - K)))

    src = jnp.concatenate(
        [box.reshape(-1), points.reshape(-1), revise.reshape(-1)])
    det, clsid = _gather(slot_i.reshape(-1), tslot_pad, src)

    detections = det.reshape(N, KP, 4)[:, :K, :]
    return detections, clsid[:, :K], topv


# speedup vs baseline: 2.7617x; 1.0317x over previous
import functools

import jax
import jax.numpy as jnp
from jax import lax
from jax.experimental import pallas as pl
from jax.experimental.pallas import tpu as pltpu
from jax.experimental.pallas import tpu_sc as plsc

N, C, H, W = 8, 80, 128, 128
HW = H * W
FL = HW * C
K = 1000
THRESH = 0.05
NB = 512
NBF = float(NB)
NCORE = 2
NSUB = 16
BPC = N // NCORE
CH = FL // NSUB
NV = CH // 16
PER_W = 1024
SLOTS = NSUB * PER_W
NCHAIN = 4
CH4 = CH // NCHAIN
NV4 = CH4 // 16
HQ = NB * 16
PQ = PER_W // NCHAIN
PCQ = (CH4 // 80) * 128
KP = 1024
PP = KP // 4
OFF_PTS = N * 4 * HW
OFF_REV = OFF_PTS + HW * 2


def _scores_body(cls_ref, center_ref, out_ref):
    s = 1.0 / (1.0 + jnp.exp(-cls_ref[...]))
    ctr = 1.0 / (1.0 + jnp.exp(-center_ref[...]))
    out_ref[...] = jnp.where(s > THRESH, s * ctr, 0.0)


def _scores(cls_hwc, center_hw1):
    BLK = 2048
    return pl.pallas_call(
        _scores_body,
        grid=(N, HW // BLK),
        in_specs=[
            pl.BlockSpec((1, BLK, C), lambda n, i: (n, i, 0)),
            pl.BlockSpec((1, BLK, 1), lambda n, i: (n, i, 0)),
        ],
        out_specs=pl.BlockSpec((1, BLK, C), lambda n, i: (n, i, 0)),
        out_shape=jax.ShapeDtypeStruct((N, HW, C), jnp.float32),
    )(cls_hwc, center_hw1)


_sel_mesh = plsc.VectorSubcoreMesh(core_axis_name="c", subcore_axis_name="s")


@functools.partial(
    pl.kernel,
    out_type=(
        jax.ShapeDtypeStruct((N, SLOTS), jnp.float32),
        jax.ShapeDtypeStruct((N, SLOTS), jnp.int32),
    ),
    mesh=_sel_mesh,
    compiler_params=pltpu.CompilerParams(needs_layout_passes=False),
    scratch_types=[
        pltpu.VMEM((CH,), jnp.float32),
        pltpu.VMEM((NCHAIN * HQ,), jnp.int32),
        pltpu.VMEM((NB,), jnp.int32),
        pltpu.VMEM((NB,), jnp.int32),
        pltpu.VMEM((PER_W,), jnp.float32),
        pltpu.VMEM((PER_W,), jnp.int32),
        pltpu.VMEM((16,), jnp.int32),
        pltpu.VMEM_SHARED((NSUB, BPC, NB), jnp.int32),
        pltpu.VMEM_SHARED((BPC, 128), jnp.int32),
    ],
)
def _select(scores_ref, slot_v_ref, slot_i_ref,
            win, hist16, histc, histt, cval, cidx, tbuf, sh_hist, sh_thr):
    cid = lax.axis_index("c")
    sid = lax.axis_index("s")
    lane = lax.iota(jnp.int32, 16)
    lane_nb = lane * NB
    ones = jnp.ones((16,), jnp.int32)
    zeros16 = jnp.zeros((16,), jnp.int32)
    nb1 = jnp.full((16,), NB - 1, jnp.int32)
    col0 = sid * CH

    def batch_hist(bl, _):
        b = cid * BPC + bl

        def zbody(i, _):
            hist16[pl.ds(i * 16, 16)] = zeros16
            return 0
        lax.fori_loop(0, NCHAIN * HQ // 16, zbody, 0)

        pltpu.sync_copy(scores_ref.at[b, pl.ds(col0, CH)], win)

        def hbody(i, _):
            for q in range(NCHAIN):
                v = win[pl.ds(q * CH4 + i * 16, 16)]
                bn = lax.min(lax.convert_element_type(v * NBF, jnp.int32),
                             nb1)
                idx = bn + lane_nb + q * HQ
                cur = plsc.load_gather(hist16, [idx])
                plsc.store_scatter(hist16, [idx], cur + ones)
            return 0
        lax.fori_loop(0, NV4, hbody, 0)

        def cbody(p, _):
            acc = hist16[pl.ds(p * 16, 16)]
            for q in range(NCHAIN):
                for l in range(16):
                    if q == 0 and l == 0:
                        continue
                    acc = acc + hist16[pl.ds(q * HQ + l * NB + p * 16, 16)]
            histc[pl.ds(p * 16, 16)] = acc
            return 0
        lax.fori_loop(0, NB // 16, cbody, 0)

        pltpu.sync_copy(histc, sh_hist.at[sid, bl])
        return 0
    lax.fori_loop(0, BPC, batch_hist, 0)
    plsc.subcore_barrier()

    @pl.when(sid < BPC)
    def _():
        bl = sid

        def zbody(i, _):
            histc[pl.ds(i * 16, 16)] = zeros16
            return 0
        lax.fori_loop(0, NB // 16, zbody, 0)

        def sum_w(wkr, _):
            pltpu.sync_copy(sh_hist.at[wkr, bl], histt)

            def abody(p, _):
                histc[pl.ds(p * 16, 16)] = (
                    histc[pl.ds(p * 16, 16)] + histt[pl.ds(p * 16, 16)])
                return 0
            lax.fori_loop(0, NB // 16, abody, 0)
            return 0
        lax.fori_loop(0, NSUB, sum_w, 0)

        def scan_bin(t, carry):
            s_after, jstar = carry
            p = NB // 16 - 1 - t
            chunk = histc[pl.ds(p * 16, 16)]
            ps = plsc.cumsum(chunk)
            total = jnp.sum(chunk)
            suffix = s_after + total - ps + chunk
            cond = suffix >= K
            pc = plsc.all_reduce_population_count(cond)[0]
            jstar = lax.select((pc > 0) & (jstar < 0),
                               p * 16 + pc - 1, jstar)
            return (s_after + total, jstar)
        _, jstar = lax.fori_loop(0, NB // 16, scan_bin,
                                 (jnp.int32(0), jnp.int32(-1)))
        jstar = lax.max(jstar, jnp.int32(0))
        tbuf[pl.ds(0, 16)] = zeros16 + jstar
        pltpu.sync_copy(tbuf, sh_thr.at[bl, pl.ds(0, 16)])
    plsc.subcore_barrier()

    neg1 = jnp.full((16,), -1.0, jnp.float32)
    hw0 = sid * (CH // 80)

    def batch_comp(bl, _):
        b = cid * BPC + bl
        pltpu.sync_copy(sh_thr.at[bl, pl.ds(0, 16)], tbuf)
        thr = tbuf[pl.ds(0, 16)][0]

        def pbody(i, _):
            cval[pl.ds(i * 16, 16)] = neg1
            return 0
        lax.fori_loop(0, PER_W // 16, pbody, 0)

        pltpu.sync_copy(scores_ref.at[b, pl.ds(col0, CH)], win)

        def cbody(i, carry):
            poss, pcb, cs = carry[:NCHAIN], carry[NCHAIN], carry[NCHAIN + 1]
            new_poss = []
            for q in range(NCHAIN):
                pos = poss[q]
                v = win[pl.ds(q * CH4 + i * 16, 16)]
                bn = lax.min(lax.convert_element_type(v * NBF, jnp.int32),
                             nb1)
                m = bn >= thr
                cnt = plsc.all_reduce_population_count(m)[0]
                ok = pos <= PQ - 16

                @pl.when(ok)
                def _(q=q, pos=pos, v=v, m=m, pcb=pcb):
                    plsc.store_compressed(cval.at[pl.ds(q * PQ + pos, 16)],
                                          v, mask=m)
                    plsc.store_compressed(cidx.at[pl.ds(q * PQ + pos, 16)],
                                          pcb + q * PCQ + lane, mask=m)

                new_poss.append(pos + lax.select(ok, cnt, jnp.int32(0)))
            wrap = cs >= 80 - 16
            pcb = pcb + lax.select(wrap, jnp.int32(64), jnp.int32(16))
            cs = lax.select(wrap, jnp.int32(0), cs + 16)
            return (*new_poss, pcb, cs)
        lax.fori_loop(0, NV4, cbody,
                      (jnp.int32(0),) * NCHAIN
                      + ((hw0 * 128).astype(jnp.int32), jnp.int32(0)))

        pltpu.sync_copy(cval, slot_v_ref.at[b, pl.ds(sid * PER_W, PER_W)])
        pltpu.sync_copy(cidx, slot_i_ref.at[b, pl.ds(sid * PER_W, PER_W)])
        return 0
    lax.fori_loop(0, BPC, batch_comp, 0)


@functools.partial(
    pl.kernel,
    out_type=(
        jax.ShapeDtypeStruct((N, KP * 4), jnp.float32),
        jax.ShapeDtypeStruct((N, KP), jnp.int32),
    ),
    mesh=_sel_mesh,
    compiler_params=pltpu.CompilerParams(needs_layout_passes=False),
    scratch_types=[
        pltpu.VMEM((PP,), jnp.int32),
        pltpu.VMEM((2, 128), jnp.int32),
        pltpu.VMEM((2, 128), jnp.int32),
        pltpu.VMEM((PP,), jnp.int32),
        pltpu.VMEM((PP,), jnp.int32),
        pltpu.VMEM((16, 128), jnp.int32),
        pltpu.VMEM((16, 128), jnp.float32),
        pltpu.VMEM((KP,), jnp.float32),
        pltpu.SemaphoreType.DMA,
    ],
)
def _gather(slot_i_ref, tslot_ref, src_ref, det_ref, cls_ref,
            tsl, asl, fid, locb, clsb, gi, gv, detv, sem):
    cid = lax.axis_index("c")
    sid = lax.axis_index("s")
    wid = sid * NCORE + cid
    b = wid // 4
    sub = wid - b * 4
    lane = lax.iota(jnp.int32, 16)

    pltpu.sync_copy(tslot_ref.at[b, pl.ds(sub * PP, PP)], tsl)

    def abody(k, _):
        h = k // 8
        off = (k - h * 8) * 16
        asl[h, pl.ds(off, 16)] = tsl[pl.ds(k * 16, 16)] + b * SLOTS
        return 0
    lax.fori_loop(0, 16, abody, 0)

    d0 = pltpu.async_copy(slot_i_ref.at[asl.at[0]], fid.at[0], sem)
    d1 = pltpu.async_copy(slot_i_ref.at[asl.at[1]], fid.at[1], sem)
    d0.wait()
    d1.wait()

    def lbody(k, _):
        h = k // 8
        off = (k - h * 8) * 16
        pc = fid[h, pl.ds(off, 16)]
        locb[pl.ds(k * 16, 16)] = lax.shift_right_logical(pc, 7)
        clsb[pl.ds(k * 16, 16)] = (pc & 127) + 1
        return 0
    lax.fori_loop(0, 16, lbody, 0)

    def make_rows(g, base_fn):
        def rbody(k, _):
            h = k // 8
            off = (k - h * 8) * 16
            loc = locb[pl.ds(k * 16, 16)]
            gi[2 * g + h, pl.ds(off, 16)] = base_fn(loc)
            return 0
        lax.fori_loop(0, 16, rbody, 0)

    box_base = b * 4 * HW
    make_rows(0, lambda loc: box_base + loc)
    make_rows(1, lambda loc: box_base + HW + loc)
    make_rows(2, lambda loc: box_base + 2 * HW + loc)
    make_rows(3, lambda loc: box_base + 3 * HW + loc)
    make_rows(4, lambda loc: OFF_PTS + loc * 2)
    make_rows(5, lambda loc: OFF_PTS + loc * 2 + 1)
    make_rows(6, lambda loc: OFF_REV + (b * HW + loc) * 2)
    make_rows(7, lambda loc: OFF_REV + (b * HW + loc) * 2 + 1)

    descs = []
    for r in range(16):
        descs.append(pltpu.async_copy(src_ref.at[gi.at[r]], gv.at[r], sem))
    for d in descs:
        d.wait()

    def dbody(k, _):
        h = k // 8
        off = (k - h * 8) * 16
        bx0 = gv[0 + h, pl.ds(off, 16)]
        bx1 = gv[2 + h, pl.ds(off, 16)]
        bx2 = gv[4 + h, pl.ds(off, 16)]
        bx3 = gv[6 + h, pl.ds(off, 16)]
        px = gv[8 + h, pl.ds(off, 16)] + gv[12 + h, pl.ds(off, 16)]
        py = gv[10 + h, pl.ds(off, 16)] + gv[14 + h, pl.ds(off, 16)]
        pidx = (k * 16 + lane) * 4
        plsc.store_scatter(detv, [pidx], px - bx0)
        plsc.store_scatter(detv, [pidx + 1], py - bx1)
        plsc.store_scatter(detv, [pidx + 2], px + bx2)
        plsc.store_scatter(detv, [pidx + 3], py + bx3)
        return 0
    lax.fori_loop(0, 16, dbody, 0)

    pltpu.sync_copy(detv, det_ref.at[b, pl.ds(sub * PP * 4, PP * 4)])
    pltpu.sync_copy(clsb, cls_ref.at[b, pl.ds(sub * PP, PP)])


def kernel(box, cls, center, points, revise):
    cls_hwc = jnp.transpose(cls, (0, 2, 3, 1)).reshape(N, HW, C)
    center_hw1 = center.reshape(N, HW, 1)
    scores = _scores(cls_hwc, center_hw1).reshape(N, FL)

    slot_v, slot_i = _select(scores)
    topv, tslot = lax.top_k(slot_v, K)
    tslot_pad = jnp.pad(tslot, ((0, 0), (0, KP - K)))

    src = jnp.concatenate(
        [box.reshape(-1), points.reshape(-1), revise.reshape(-1)])
    det, clsid = _gather(slot_i.reshape(-1), tslot_pad, src)

    detections = det.reshape(N, KP, 4)[:, :K, :]
    return detections, clsid[:, :K], topv


# speedup vs baseline: 2.7851x; 1.0085x over previous
import functools

import jax
import jax.numpy as jnp
from jax import lax
from jax.experimental import pallas as pl
from jax.experimental.pallas import tpu as pltpu
from jax.experimental.pallas import tpu_sc as plsc

N, C, H, W = 8, 80, 128, 128
HW = H * W
FL = HW * C
K = 1000
THRESH = 0.05
NB = 512
NBF = float(NB)
NCORE = 2
NSUB = 16
BPC = N // NCORE
CH = FL // NSUB
NV = CH // 16
PER_W = 512
SLOTS = NSUB * PER_W
NCHAIN = 4
CH4 = CH // NCHAIN
NV4 = CH4 // 16
HQ = NB * 16
PQ = PER_W // NCHAIN
PCQ = (CH4 // 80) * 128
KP = 1024
PP = KP // 4
OFF_PTS = N * 4 * HW
OFF_REV = OFF_PTS + HW * 2


def _scores_body(cls_ref, center_ref, out_ref):
    s = 1.0 / (1.0 + jnp.exp(-cls_ref[0]))
    ident = (lax.broadcasted_iota(jnp.int32, (C, C), 0)
             == lax.broadcasted_iota(jnp.int32, (C, C), 1)
             ).astype(jnp.float32)
    st = lax.dot_general(s, ident, (((0,), (0,)), ((), ())),
                         precision=lax.Precision.HIGHEST,
                         preferred_element_type=jnp.float32)
    ctr = 1.0 / (1.0 + jnp.exp(-center_ref[0]))
    out_ref[0] = jnp.where(st > THRESH, st * ctr, 0.0)


def _scores(cls_chw, center_hw1):
    BLK = 2048
    return pl.pallas_call(
        _scores_body,
        grid=(N, HW // BLK),
        in_specs=[
            pl.BlockSpec((1, C, BLK), lambda n, i: (n, 0, i)),
            pl.BlockSpec((1, BLK, 1), lambda n, i: (n, i, 0)),
        ],
        out_specs=pl.BlockSpec((1, BLK, C), lambda n, i: (n, i, 0)),
        out_shape=jax.ShapeDtypeStruct((N, HW, C), jnp.float32),
    )(cls_chw, center_hw1)


_sel_mesh = plsc.VectorSubcoreMesh(core_axis_name="c", subcore_axis_name="s")


@functools.partial(
    pl.kernel,
    out_type=(
        jax.ShapeDtypeStruct((N, SLOTS), jnp.float32),
        jax.ShapeDtypeStruct((N, SLOTS), jnp.int32),
    ),
    mesh=_sel_mesh,
    compiler_params=pltpu.CompilerParams(needs_layout_passes=False),
    scratch_types=[
        pltpu.VMEM((CH,), jnp.float32),
        pltpu.VMEM((NCHAIN * HQ,), jnp.int32),
        pltpu.VMEM((NB,), jnp.int32),
        pltpu.VMEM((NB,), jnp.int32),
        pltpu.VMEM((PER_W,), jnp.float32),
        pltpu.VMEM((PER_W,), jnp.int32),
        pltpu.VMEM((16,), jnp.int32),
        pltpu.VMEM_SHARED((NSUB, BPC, NB), jnp.int32),
        pltpu.VMEM_SHARED((BPC, 128), jnp.int32),
    ],
)
def _select(scores_ref, slot_v_ref, slot_i_ref,
            win, hist16, histc, histt, cval, cidx, tbuf, sh_hist, sh_thr):
    cid = lax.axis_index("c")
    sid = lax.axis_index("s")
    lane = lax.iota(jnp.int32, 16)
    lane_nb = lane * NB
    ones = jnp.ones((16,), jnp.int32)
    zeros16 = jnp.zeros((16,), jnp.int32)
    nb1 = jnp.full((16,), NB - 1, jnp.int32)
    col0 = sid * CH

    def batch_hist(bl, _):
        b = cid * BPC + bl

        def zbody(i, _):
            hist16[pl.ds(i * 16, 16)] = zeros16
            return 0
        lax.fori_loop(0, NCHAIN * HQ // 16, zbody, 0)

        pltpu.sync_copy(scores_ref.at[b, pl.ds(col0, CH)], win)

        def hbody(i, _):
            for q in range(NCHAIN):
                v = win[pl.ds(q * CH4 + i * 16, 16)]
                bn = lax.min(lax.convert_element_type(v * NBF, jnp.int32),
                             nb1)
                idx = bn + lane_nb + q * HQ
                cur = plsc.load_gather(hist16, [idx])
                plsc.store_scatter(hist16, [idx], cur + ones)
            return 0
        lax.fori_loop(0, NV4, hbody, 0)

        def cbody(p, _):
            acc = hist16[pl.ds(p * 16, 16)]
            for q in range(NCHAIN):
                for l in range(16):
                    if q == 0 and l == 0:
                        continue
                    acc = acc + hist16[pl.ds(q * HQ + l * NB + p * 16, 16)]
            histc[pl.ds(p * 16, 16)] = acc
            return 0
        lax.fori_loop(0, NB // 16, cbody, 0)

        pltpu.sync_copy(histc, sh_hist.at[sid, bl])
        return 0
    lax.fori_loop(0, BPC, batch_hist, 0)
    plsc.subcore_barrier()

    @pl.when(sid < BPC)
    def _():
        bl = sid

        def zbody(i, _):
            histc[pl.ds(i * 16, 16)] = zeros16
            return 0
        lax.fori_loop(0, NB // 16, zbody, 0)

        def sum_w(wkr, _):
            pltpu.sync_copy(sh_hist.at[wkr, bl], histt)

            def abody(p, _):
                histc[pl.ds(p * 16, 16)] = (
                    histc[pl.ds(p * 16, 16)] + histt[pl.ds(p * 16, 16)])
                return 0
            lax.fori_loop(0, NB // 16, abody, 0)
            return 0
        lax.fori_loop(0, NSUB, sum_w, 0)

        def scan_bin(t, carry):
            s_after, jstar = carry
            p = NB // 16 - 1 - t
            chunk = histc[pl.ds(p * 16, 16)]
            ps = plsc.cumsum(chunk)
            total = jnp.sum(chunk)
            suffix = s_after + total - ps + chunk
            cond = suffix >= K
            pc = plsc.all_reduce_population_count(cond)[0]
            jstar = lax.select((pc > 0) & (jstar < 0),
                               p * 16 + pc - 1, jstar)
            return (s_after + total, jstar)
        _, jstar = lax.fori_loop(0, NB // 16, scan_bin,
                                 (jnp.int32(0), jnp.int32(-1)))
        jstar = lax.max(jstar, jnp.int32(0))
        tbuf[pl.ds(0, 16)] = zeros16 + jstar
        pltpu.sync_copy(tbuf, sh_thr.at[bl, pl.ds(0, 16)])
    plsc.subcore_barrier()

    neg1 = jnp.full((16,), -1.0, jnp.float32)
    hw0 = sid * (CH // 80)

    def batch_comp(bl, _):
        b = cid * BPC + bl
        pltpu.sync_copy(sh_thr.at[bl, pl.ds(0, 16)], tbuf)
        thr = tbuf[pl.ds(0, 16)][0]

        def pbody(i, _):
            cval[pl.ds(i * 16, 16)] = neg1
            return 0
        lax.fori_loop(0, PER_W // 16, pbody, 0)

        pltpu.sync_copy(scores_ref.at[b, pl.ds(col0, CH)], win)

        def cbody(i, carry):
            poss, pcb, cs = carry[:NCHAIN], carry[NCHAIN], carry[NCHAIN + 1]
            new_poss = []
            for q in range(NCHAIN):
                pos = poss[q]
                v = win[pl.ds(q * CH4 + i * 16, 16)]
                bn = lax.min(lax.convert_element_type(v * NBF, jnp.int32),
                             nb1)
                m = bn >= thr
                cnt = plsc.all_reduce_population_count(m)[0]
                ok = pos <= PQ - 16

                @pl.when(ok)
                def _(q=q, pos=pos, v=v, m=m, pcb=pcb):
                    plsc.store_compressed(cval.at[pl.ds(q * PQ + pos, 16)],
                                          v, mask=m)
                    plsc.store_compressed(cidx.at[pl.ds(q * PQ + pos, 16)],
                                          pcb + q * PCQ + lane, mask=m)

                new_poss.append(pos + lax.select(ok, cnt, jnp.int32(0)))
            wrap = cs >= 80 - 16
            pcb = pcb + lax.select(wrap, jnp.int32(64), jnp.int32(16))
            cs = lax.select(wrap, jnp.int32(0), cs + 16)
            return (*new_poss, pcb, cs)
        lax.fori_loop(0, NV4, cbody,
                      (jnp.int32(0),) * NCHAIN
                      + ((hw0 * 128).astype(jnp.int32), jnp.int32(0)))

        pltpu.sync_copy(cval, slot_v_ref.at[b, pl.ds(sid * PER_W, PER_W)])
        pltpu.sync_copy(cidx, slot_i_ref.at[b, pl.ds(sid * PER_W, PER_W)])
        return 0
    lax.fori_loop(0, BPC, batch_comp, 0)


@functools.partial(
    pl.kernel,
    out_type=(
        jax.ShapeDtypeStruct((N, KP * 4), jnp.float32),
        jax.ShapeDtypeStruct((N, KP), jnp.int32),
    ),
    mesh=_sel_mesh,
    compiler_params=pltpu.CompilerParams(needs_layout_passes=False),
    scratch_types=[
        pltpu.VMEM((PP,), jnp.int32),
        pltpu.VMEM((2, 128), jnp.int32),
        pltpu.VMEM((2, 128), jnp.int32),
        pltpu.VMEM((PP,), jnp.int32),
        pltpu.VMEM((PP,), jnp.int32),
        pltpu.VMEM((16, 128), jnp.int32),
        pltpu.VMEM((16, 128), jnp.float32),
        pltpu.VMEM((KP,), jnp.float32),
        pltpu.SemaphoreType.DMA,
    ],
)
def _gather(slot_i_ref, tslot_ref, src_ref, det_ref, cls_ref,
            tsl, asl, fid, locb, clsb, gi, gv, detv, sem):
    cid = lax.axis_index("c")
    sid = lax.axis_index("s")
    wid = sid * NCORE + cid
    b = wid // 4
    sub = wid - b * 4
    lane = lax.iota(jnp.int32, 16)

    pltpu.sync_copy(tslot_ref.at[b, pl.ds(sub * PP, PP)], tsl)

    def abody(k, _):
        h = k // 8
        off = (k - h * 8) * 16
        asl[h, pl.ds(off, 16)] = tsl[pl.ds(k * 16, 16)] + b * SLOTS
        return 0
    lax.fori_loop(0, 16, abody, 0)

    d0 = pltpu.async_copy(slot_i_ref.at[asl.at[0]], fid.at[0], sem)
    d1 = pltpu.async_copy(slot_i_ref.at[asl.at[1]], fid.at[1], sem)
    d0.wait()
    d1.wait()

    def lbody(k, _):
        h = k // 8
        off = (k - h * 8) * 16
        pc = fid[h, pl.ds(off, 16)]
        locb[pl.ds(k * 16, 16)] = lax.shift_right_logical(pc, 7)
        clsb[pl.ds(k * 16, 16)] = (pc & 127) + 1
        return 0
    lax.fori_loop(0, 16, lbody, 0)

    def make_rows(g, base_fn):
        def rbody(k, _):
            h = k // 8
            off = (k - h * 8) * 16
            loc = locb[pl.ds(k * 16, 16)]
            gi[2 * g + h, pl.ds(off, 16)] = base_fn(loc)
            return 0
        lax.fori_loop(0, 16, rbody, 0)

    box_base = b * 4 * HW
    make_rows(0, lambda loc: box_base + loc)
    make_rows(1, lambda loc: box_base + HW + loc)
    make_rows(2, lambda loc: box_base + 2 * HW + loc)
    make_rows(3, lambda loc: box_base + 3 * HW + loc)
    make_rows(4, lambda loc: OFF_PTS + loc * 2)
    make_rows(5, lambda loc: OFF_PTS + loc * 2 + 1)
    make_rows(6, lambda loc: OFF_REV + (b * HW + loc) * 2)
    make_rows(7, lambda loc: OFF_REV + (b * HW + loc) * 2 + 1)

    descs = []
    for r in range(16):
        descs.append(pltpu.async_copy(src_ref.at[gi.at[r]], gv.at[r], sem))
    for d in descs:
        d.wait()

    def dbody(k, _):
        h = k // 8
        off = (k - h * 8) * 16
        bx0 = gv[0 + h, pl.ds(off, 16)]
        bx1 = gv[2 + h, pl.ds(off, 16)]
        bx2 = gv[4 + h, pl.ds(off, 16)]
        bx3 = gv[6 + h, pl.ds(off, 16)]
        px = gv[8 + h, pl.ds(off, 16)] + gv[12 + h, pl.ds(off, 16)]
        py = gv[10 + h, pl.ds(off, 16)] + gv[14 + h, pl.ds(off, 16)]
        pidx = (k * 16 + lane) * 4
        plsc.store_scatter(detv, [pidx], px - bx0)
        plsc.store_scatter(detv, [pidx + 1], py - bx1)
        plsc.store_scatter(detv, [pidx + 2], px + bx2)
        plsc.store_scatter(detv, [pidx + 3], py + bx3)
        return 0
    lax.fori_loop(0, 16, dbody, 0)

    pltpu.sync_copy(detv, det_ref.at[b, pl.ds(sub * PP * 4, PP * 4)])
    pltpu.sync_copy(clsb, cls_ref.at[b, pl.ds(sub * PP, PP)])


def kernel(box, cls, center, points, revise):
    cls_chw = cls.reshape(N, C, HW)
    center_hw1 = center.reshape(N, HW, 1)
    scores = _scores(cls_chw, center_hw1).reshape(N, FL)

    slot_v, slot_i = _select(scores)
    topv, tslot = lax.top_k(slot_v, K)
    tslot_pad = jnp.pad(tslot, ((0, 0), (0, KP - K)))

    src = jnp.concatenate(
        [box.reshape(-1), points.reshape(-1), revise.reshape(-1)])
    det, clsid = _gather(slot_i.reshape(-1), tslot_pad, src)

    detections = det.reshape(N, KP, 4)[:, :K, :]
    return detections, clsid[:, :K], topv


# speedup vs baseline: 3.3709x; 1.2103x over previous
import functools

import jax
import jax.numpy as jnp
from jax import lax
from jax.experimental import pallas as pl
from jax.experimental.pallas import tpu as pltpu
from jax.experimental.pallas import tpu_sc as plsc

N, C, H, W = 8, 80, 128, 128
HW = H * W
FL = HW * C
K = 1000
THRESH = 0.05
NB = 512
NBF = float(NB)
NCORE = 2
NSUB = 16
BPC = N // NCORE
CH = FL // NSUB
PER_W = 1024
SLOTS = NSUB * PER_W
CP = 128
FLP = HW * CP
CHP_TOT = FLP // NSUB
NPIECE = 4
CHP = CHP_TOT // NPIECE
NVP = CHP // 16
HQ = NB * 16
PQ = PER_W // NPIECE
KP = 1024
PP = KP // 4
OFF_PTS = N * 4 * HW
OFF_REV = OFF_PTS + HW * 2


def _scores_body(cls_ref, center_ref, out_ref):
    s = 1.0 / (1.0 + jnp.exp(-cls_ref[0]))
    ident = (lax.broadcasted_iota(jnp.int32, (C, CP), 0)
             == lax.broadcasted_iota(jnp.int32, (C, CP), 1)
             ).astype(jnp.float32)
    st = lax.dot_general(s, ident, (((0,), (0,)), ((), ())),
                         precision=lax.Precision.HIGHEST,
                         preferred_element_type=jnp.float32)
    ctr = 1.0 / (1.0 + jnp.exp(-center_ref[0]))
    out_ref[0] = jnp.where(st > THRESH, st * ctr, 0.0)


def _scores(cls_chw, center_hw1):
    BLK = 2048
    return pl.pallas_call(
        _scores_body,
        grid=(N, HW // BLK),
        in_specs=[
            pl.BlockSpec((1, C, BLK), lambda n, i: (n, 0, i)),
            pl.BlockSpec((1, BLK, 1), lambda n, i: (n, i, 0)),
        ],
        out_specs=pl.BlockSpec((1, BLK, CP), lambda n, i: (n, i, 0)),
        out_shape=jax.ShapeDtypeStruct((N, HW, CP), jnp.float32),
    )(cls_chw, center_hw1)


_sel_mesh = plsc.VectorSubcoreMesh(core_axis_name="c", subcore_axis_name="s")


@functools.partial(
    pl.kernel,
    out_type=(
        jax.ShapeDtypeStruct((N, SLOTS), jnp.float32),
        jax.ShapeDtypeStruct((N, SLOTS), jnp.int32),
    ),
    mesh=_sel_mesh,
    compiler_params=pltpu.CompilerParams(needs_layout_passes=False),
    scratch_types=[
        pltpu.VMEM((CHP,), jnp.float32),
        pltpu.VMEM((HQ,), jnp.int32),
        pltpu.VMEM((NB,), jnp.int32),
        pltpu.VMEM((NB,), jnp.int32),
        pltpu.VMEM((PER_W,), jnp.float32),
        pltpu.VMEM((PER_W,), jnp.int32),
        pltpu.VMEM((16,), jnp.int32),
        pltpu.VMEM_SHARED((NSUB, BPC, NB), jnp.int32),
        pltpu.VMEM_SHARED((BPC, 128), jnp.int32),
    ],
)
def _select(scores_ref, slot_v_ref, slot_i_ref,
            win, hist16, histc, histt, cval, cidx, tbuf, sh_hist, sh_thr):
    cid = lax.axis_index("c")
    sid = lax.axis_index("s")
    lane = lax.iota(jnp.int32, 16)
    lane_nb = lane * NB
    ones = jnp.ones((16,), jnp.int32)
    zeros16 = jnp.zeros((16,), jnp.int32)
    nb1 = jnp.full((16,), NB - 1, jnp.int32)
    col0 = sid * CHP_TOT

    def batch_hist(bl, _):
        b = cid * BPC + bl

        def zbody(i, _):
            hist16[pl.ds(i * 16, 16)] = zeros16
            return 0
        lax.fori_loop(0, HQ // 16, zbody, 0)

        def piece(p, _):
            pltpu.sync_copy(scores_ref.at[b, pl.ds(col0 + p * CHP, CHP)],
                            win)

            def hbody(i, _):
                v = win[pl.ds(i * 16, 16)]
                bn = lax.min(lax.convert_element_type(v * NBF, jnp.int32),
                             nb1)
                idx = bn + lane_nb
                cur = plsc.load_gather(hist16, [idx])
                plsc.store_scatter(hist16, [idx], cur + ones)
                return 0
            lax.fori_loop(0, NVP, hbody, 0)
            return 0
        lax.fori_loop(0, NPIECE, piece, 0)

        def cbody(p, _):
            acc = hist16[pl.ds(p * 16, 16)]
            for l in range(1, 16):
                acc = acc + hist16[pl.ds(l * NB + p * 16, 16)]
            histc[pl.ds(p * 16, 16)] = acc
            return 0
        lax.fori_loop(0, NB // 16, cbody, 0)

        pltpu.sync_copy(histc, sh_hist.at[sid, bl])
        return 0
    lax.fori_loop(0, BPC, batch_hist, 0)
    plsc.subcore_barrier()

    @pl.when(sid < BPC)
    def _():
        bl = sid

        def zbody(i, _):
            histc[pl.ds(i * 16, 16)] = zeros16
            return 0
        lax.fori_loop(0, NB // 16, zbody, 0)

        def sum_w(wkr, _):
            pltpu.sync_copy(sh_hist.at[wkr, bl], histt)

            def abody(p, _):
                histc[pl.ds(p * 16, 16)] = (
                    histc[pl.ds(p * 16, 16)] + histt[pl.ds(p * 16, 16)])
                return 0
            lax.fori_loop(0, NB // 16, abody, 0)
            return 0
        lax.fori_loop(0, NSUB, sum_w, 0)

        def scan_bin(t, carry):
            s_after, jstar = carry
            p = NB // 16 - 1 - t
            chunk = histc[pl.ds(p * 16, 16)]
            ps = plsc.cumsum(chunk)
            total = jnp.sum(chunk)
            suffix = s_after + total - ps + chunk
            cond = suffix >= K
            pc = plsc.all_reduce_population_count(cond)[0]
            jstar = lax.select((pc > 0) & (jstar < 0),
                               p * 16 + pc - 1, jstar)
            return (s_after + total, jstar)
        _, jstar = lax.fori_loop(0, NB // 16, scan_bin,
                                 (jnp.int32(0), jnp.int32(-1)))
        jstar = lax.max(jstar, jnp.int32(0))
        tbuf[pl.ds(0, 16)] = zeros16 + jstar
        pltpu.sync_copy(tbuf, sh_thr.at[bl, pl.ds(0, 16)])
    plsc.subcore_barrier()

    neg1 = jnp.full((16,), -1.0, jnp.float32)

    def batch_comp(bl, _):
        b = cid * BPC + bl
        pltpu.sync_copy(sh_thr.at[bl, pl.ds(0, 16)], tbuf)
        thr = tbuf[pl.ds(0, 16)][0]

        def pbody(i, _):
            cval[pl.ds(i * 16, 16)] = neg1
            return 0
        lax.fori_loop(0, PER_W // 16, pbody, 0)

        def piece(p, _):
            pltpu.sync_copy(scores_ref.at[b, pl.ds(col0 + p * CHP, CHP)],
                            win)
            base_pc = col0 + p * CHP

            def cbody(i, pos):
                v = win[pl.ds(i * 16, 16)]
                bn = lax.min(lax.convert_element_type(v * NBF, jnp.int32),
                             nb1)
                m = bn >= thr
                cnt = plsc.all_reduce_population_count(m)[0]
                ok = pos <= PQ - 16

                @pl.when(ok)
                def _():
                    plsc.store_compressed(cval.at[pl.ds(p * PQ + pos, 16)],
                                          v, mask=m)
                    plsc.store_compressed(cidx.at[pl.ds(p * PQ + pos, 16)],
                                          base_pc + i * 16 + lane, mask=m)

                return pos + lax.select(ok, cnt, jnp.int32(0))
            lax.fori_loop(0, NVP, cbody, jnp.int32(0))
            return 0
        lax.fori_loop(0, NPIECE, piece, 0)

        pltpu.sync_copy(cval, slot_v_ref.at[b, pl.ds(sid * PER_W, PER_W)])
        pltpu.sync_copy(cidx, slot_i_ref.at[b, pl.ds(sid * PER_W, PER_W)])
        return 0
    lax.fori_loop(0, BPC, batch_comp, 0)


@functools.partial(
    pl.kernel,
    out_type=(
        jax.ShapeDtypeStruct((N, KP * 4), jnp.float32),
        jax.ShapeDtypeStruct((N, KP), jnp.int32),
    ),
    mesh=_sel_mesh,
    compiler_params=pltpu.CompilerParams(needs_layout_passes=False),
    scratch_types=[
        pltpu.VMEM((PP,), jnp.int32),
        pltpu.VMEM((2, 128), jnp.int32),
        pltpu.VMEM((2, 128), jnp.int32),
        pltpu.VMEM((PP,), jnp.int32),
        pltpu.VMEM((PP,), jnp.int32),
        pltpu.VMEM((16, 128), jnp.int32),
        pltpu.VMEM((16, 128), jnp.float32),
        pltpu.VMEM((KP,), jnp.float32),
        pltpu.SemaphoreType.DMA,
    ],
)
def _gather(slot_i_ref, tslot_ref, src_ref, det_ref, cls_ref,
            tsl, asl, fid, locb, clsb, gi, gv, detv, sem):
    cid = lax.axis_index("c")
    sid = lax.axis_index("s")
    wid = sid * NCORE + cid
    b = wid // 4
    sub = wid - b * 4
    lane = lax.iota(jnp.int32, 16)

    pltpu.sync_copy(tslot_ref.at[b, pl.ds(sub * PP, PP)], tsl)

    def abody(k, _):
        h = k // 8
        off = (k - h * 8) * 16
        asl[h, pl.ds(off, 16)] = tsl[pl.ds(k * 16, 16)] + b * SLOTS
        return 0
    lax.fori_loop(0, 16, abody, 0)

    d0 = pltpu.async_copy(slot_i_ref.at[asl.at[0]], fid.at[0], sem)
    d1 = pltpu.async_copy(slot_i_ref.at[asl.at[1]], fid.at[1], sem)
    d0.wait()
    d1.wait()

    def lbody(k, _):
        h = k // 8
        off = (k - h * 8) * 16
        pc = fid[h, pl.ds(off, 16)]
        locb[pl.ds(k * 16, 16)] = lax.shift_right_logical(pc, 7)
        clsb[pl.ds(k * 16, 16)] = (pc & 127) + 1
        return 0
    lax.fori_loop(0, 16, lbody, 0)

    def make_rows(g, base_fn):
        def rbody(k, _):
            h = k // 8
            off = (k - h * 8) * 16
            loc = locb[pl.ds(k * 16, 16)]
            gi[2 * g + h, pl.ds(off, 16)] = base_fn(loc)
            return 0
        lax.fori_loop(0, 16, rbody, 0)

    box_base = b * 4 * HW
    make_rows(0, lambda loc: box_base + loc)
    make_rows(1, lambda loc: box_base + HW + loc)
    make_rows(2, lambda loc: box_base + 2 * HW + loc)
    make_rows(3, lambda loc: box_base + 3 * HW + loc)
    make_rows(4, lambda loc: OFF_PTS + loc * 2)
    make_rows(5, lambda loc: OFF_PTS + loc * 2 + 1)
    make_rows(6, lambda loc: OFF_REV + (b * HW + loc) * 2)
    make_rows(7, lambda loc: OFF_REV + (b * HW + loc) * 2 + 1)

    descs = []
    for r in range(16):
        descs.append(pltpu.async_copy(src_ref.at[gi.at[r]], gv.at[r], sem))
    for d in descs:
        d.wait()

    def dbody(k, _):
        h = k // 8
        off = (k - h * 8) * 16
        bx0 = gv[0 + h, pl.ds(off, 16)]
        bx1 = gv[2 + h, pl.ds(off, 16)]
        bx2 = gv[4 + h, pl.ds(off, 16)]
        bx3 = gv[6 + h, pl.ds(off, 16)]
        px = gv[8 + h, pl.ds(off, 16)] + gv[12 + h, pl.ds(off, 16)]
        py = gv[10 + h, pl.ds(off, 16)] + gv[14 + h, pl.ds(off, 16)]
        pidx = (k * 16 + lane) * 4
        plsc.store_scatter(detv, [pidx], px - bx0)
        plsc.store_scatter(detv, [pidx + 1], py - bx1)
        plsc.store_scatter(detv, [pidx + 2], px + bx2)
        plsc.store_scatter(detv, [pidx + 3], py + bx3)
        return 0
    lax.fori_loop(0, 16, dbody, 0)

    pltpu.sync_copy(detv, det_ref.at[b, pl.ds(sub * PP * 4, PP * 4)])
    pltpu.sync_copy(clsb, cls_ref.at[b, pl.ds(sub * PP, PP)])


def kernel(box, cls, center, points, revise):
    cls_chw = cls.reshape(N, C, HW)
    center_hw1 = center.reshape(N, HW, 1)
    scores = _scores(cls_chw, center_hw1).reshape(N, FLP)

    slot_v, slot_i = _select(scores)
    topv, tslot = lax.top_k(slot_v, K)
    tslot_pad = jnp.pad(tslot, ((0, 0), (0, KP - K)))

    src = jnp.concatenate(
        [box.reshape(-1), points.reshape(-1), revise.reshape(-1)])
    det, clsid = _gather(slot_i.reshape(-1), tslot_pad, src)

    detections = det.reshape(N, KP, 4)[:, :K, :]
    return detections, clsid[:, :K], topv
